# Initial kernel scaffold; baseline (speedup 1.0000x reference)
import jax
import jax.numpy as jnp
from jax.experimental import pallas as pl


def kernel(x, pos, edge_index, ppW1, ppb1, aW1, ab1, gW1, gb1, ppW2, ppb2, aW2, ab2, gW2, gb2, scW, scb):
    raise NotImplementedError("write your pallas kernel here")


# speedup vs baseline: 2.7016x; 2.7016x over previous
import functools

import jax
import jax.numpy as jnp
from jax import lax
from jax.experimental import pallas as pl
from jax.experimental.pallas import tpu as pltpu
from jax.experimental.pallas import tpu_sc as plsc

NC = 2
NS = 16
NW = NC * NS
CH = 128

_f32 = jnp.float32


def _sc_gather_pos(pos16, src, dst):
  e = src.shape[0]
  nchunk = e // CH
  nk = (nchunk + NW - 1) // NW
  mesh = plsc.VectorSubcoreMesh(core_axis_name="c", subcore_axis_name="s")

  @functools.partial(
      pl.kernel,
      out_type=[
          jax.ShapeDtypeStruct((e, 16), _f32),
          jax.ShapeDtypeStruct((e, 16), _f32),
      ],
      mesh=mesh,
      scratch_types=[
          pltpu.VMEM((CH,), jnp.int32),
          pltpu.VMEM((CH,), jnp.int32),
          pltpu.VMEM((CH, 16), _f32),
          pltpu.VMEM((CH, 16), _f32),
          pltpu.SemaphoreType.DMA,
      ],
      compiler_params=pltpu.CompilerParams(use_tc_tiling_on_sc=False),
  )
  def gk(pos_hbm, src_hbm, dst_hbm, pj_hbm, pi_hbm,
         idx_s, idx_d, rpj, rpi, sem):
    wid = lax.axis_index("s") * NC + lax.axis_index("c")

    def body(k, carry):
      c = wid + k * NW

      @pl.when(c < nchunk)
      def _():
        base = c * CH
        pltpu.sync_copy(src_hbm.at[pl.ds(base, CH)], idx_s)
        pltpu.sync_copy(dst_hbm.at[pl.ds(base, CH)], idx_d)
        cb = pltpu.async_copy(pos_hbm.at[idx_s], rpj, sem)
        cc = pltpu.async_copy(pos_hbm.at[idx_d], rpi, sem)
        cb.wait()
        cc.wait()
        pltpu.sync_copy(rpj, pj_hbm.at[pl.ds(base, CH)])
        pltpu.sync_copy(rpi, pi_hbm.at[pl.ds(base, CH)])

      return carry

    lax.fori_loop(0, nk, body, 0)

  return gk(pos16, src, dst)


def _sc_gather1(x, src):
  n, d = x.shape
  e = src.shape[0]
  nchunk = e // CH
  nk = (nchunk + NW - 1) // NW
  mesh = plsc.VectorSubcoreMesh(core_axis_name="c", subcore_axis_name="s")

  @functools.partial(
      pl.kernel,
      out_type=jax.ShapeDtypeStruct((e, d), _f32),
      mesh=mesh,
      scratch_types=[
          pltpu.VMEM((CH,), jnp.int32),
          pltpu.VMEM((CH, d), _f32),
          pltpu.SemaphoreType.DMA,
      ],
  )
  def gk(x_hbm, src_hbm, xj_hbm, idx_s, rx, sem):
    wid = lax.axis_index("s") * NC + lax.axis_index("c")

    def body(k, carry):
      c = wid + k * NW

      @pl.when(c < nchunk)
      def _():
        base = c * CH
        pltpu.sync_copy(src_hbm.at[pl.ds(base, CH)], idx_s)
        pltpu.async_copy(x_hbm.at[idx_s], rx, sem).wait()
        pltpu.sync_copy(rx, xj_hbm.at[pl.ds(base, CH)])

      return carry

    lax.fori_loop(0, nk, body, 0)

  return gk(x, src)


def _sc_scatter1(msg, dst, n, dep=None):
  if dep is None:
    dep = jnp.zeros((8,), jnp.int32)
  e, da = msg.shape
  nchunk = e // CH
  nk = (nchunk + NW - 1) // NW
  br = (n // NS) // 8 * 8
  tail = n - br * NS
  mesh = plsc.VectorSubcoreMesh(core_axis_name="c", subcore_axis_name="s")

  zeros_a = jnp.zeros((max(br, tail), da), _f32)

  @functools.partial(
      pl.kernel,
      out_type=jax.ShapeDtypeStruct((NC, n, da), _f32),
      mesh=mesh,
      scratch_types=[
          pltpu.VMEM((CH,), jnp.int32),
          pltpu.VMEM((CH, da), _f32),
          pltpu.VMEM_SHARED((n, da), _f32),
          pltpu.SemaphoreType.DMA,
      ],
      compiler_params=pltpu.CompilerParams(
          use_tc_tiling_on_sc=(da % 128 == 0)),
  )
  def sk(msg_hbm, dst_hbm, za_hbm, dep_hbm, pa_hbm, idx, ba, acc_a, sem):
    del dep_hbm
    cid = lax.axis_index("c")
    sid = lax.axis_index("s")
    wid = sid * NC + cid
    r0 = sid * br
    pltpu.sync_copy(za_hbm.at[pl.ds(0, br)], acc_a.at[pl.ds(r0, br)])

    @pl.when(sid == NS - 1)
    def _():
      pltpu.sync_copy(za_hbm.at[pl.ds(0, tail)],
                      acc_a.at[pl.ds(NS * br, tail)])

    plsc.subcore_barrier()

    def body(k, carry):
      c = wid + k * NW

      @pl.when(c < nchunk)
      def _():
        base = c * CH
        pltpu.sync_copy(dst_hbm.at[pl.ds(base, CH)], idx)
        pltpu.async_copy(msg_hbm.at[pl.ds(base, CH)], ba, sem).wait()
        pltpu.sync_copy(ba, acc_a.at[idx], add=True)

      return carry

    lax.fori_loop(0, nk, body, 0)
    plsc.subcore_barrier()
    pltpu.sync_copy(acc_a.at[pl.ds(r0, br)], pa_hbm.at[cid, pl.ds(r0, br)])

    @pl.when(sid == NS - 1)
    def _():
      pltpu.sync_copy(acc_a.at[pl.ds(NS * br, tail)],
                      pa_hbm.at[cid, pl.ds(NS * br, tail)])

  return sk(msg, dst, zeros_a, dep)


def _sc_scatter(msg_a, msg_b, dst, n):
  pa = _sc_scatter1(msg_a, dst, n)
  pb = _sc_scatter1(msg_b, dst, n, dep=pa)
  return pa, pb


def _edge_body(xj_ref, pi_ref, pj_ref, wpi_ref, wpj_ref, w9_ref, ppb_ref,
               awa_ref, awb_ref, ab_ref, oa_ref, ob_ref):
  xj = xj_ref[...]
  pi = pi_ref[...]
  pj = pj_ref[...]
  v = pi - pj
  dij = jnp.sqrt(jnp.sum(v * v, axis=1, keepdims=True))
  r = pi @ wpi_ref[...] + pj @ wpj_ref[...] + dij * w9_ref[...] + ppb_ref[...]
  r = jnp.maximum(r, 0.0)
  g = xj @ awa_ref[...] + r @ awb_ref[...] + ab_ref[...]
  g = jnp.maximum(g, 0.0)
  m = jnp.max(g, axis=1, keepdims=True)
  eg = jnp.exp(g - m)
  s = eg / jnp.sum(eg, axis=1, keepdims=True)
  da = xj.shape[1]
  oa_ref[...] = s[:, :da] * xj
  ob_ref[...] = s[:, da:] * r


def _tc_edge(xj, pi16, pj16, wpi, wpj, w9, ppb, awa, awb, ab, block=1000):
  e, d = xj.shape
  dp = wpi.shape[1]
  df = d + dp
  grid = e // block
  full = lambda i: (0, 0)
  return pl.pallas_call(
      _edge_body,
      grid=(grid,),
      in_specs=[
          pl.BlockSpec((block, d), lambda i: (i, 0)),
          pl.BlockSpec((block, 16), lambda i: (i, 0)),
          pl.BlockSpec((block, 16), lambda i: (i, 0)),
          pl.BlockSpec((16, dp), full),
          pl.BlockSpec((16, dp), full),
          pl.BlockSpec((1, dp), full),
          pl.BlockSpec((1, dp), full),
          pl.BlockSpec((d, df), full),
          pl.BlockSpec((dp, df), full),
          pl.BlockSpec((1, df), full),
      ],
      out_specs=[
          pl.BlockSpec((block, d), lambda i: (i, 0)),
          pl.BlockSpec((block, dp), lambda i: (i, 0)),
      ],
      out_shape=[
          jax.ShapeDtypeStruct((e, d), _f32),
          jax.ShapeDtypeStruct((e, dp), _f32),
      ],
  )(xj, pi16, pj16, wpi, wpj, w9, ppb, awa, awb, ab)


def _global_body(pa_ref, pb_ref, gwa_ref, gwb_ref, gb_ref, o_ref):
  a = pa_ref[0] + pa_ref[1]
  b = pb_ref[0] + pb_ref[1]
  o_ref[...] = jnp.maximum(a @ gwa_ref[...] + b @ gwb_ref[...] + gb_ref[...],
                           0.0)


def _tc_global(pa, pb, gwa, gwb, gb, block=1000):
  _, n, da = pa.shape
  db = pb.shape[2]
  dout = gwa.shape[1]
  grid = n // block
  full = lambda i: (0, 0)
  return pl.pallas_call(
      _global_body,
      grid=(grid,),
      in_specs=[
          pl.BlockSpec((NC, block, da), lambda i: (0, i, 0)),
          pl.BlockSpec((NC, block, db), lambda i: (0, i, 0)),
          pl.BlockSpec((da, dout), full),
          pl.BlockSpec((db, dout), full),
          pl.BlockSpec((1, dout), full),
      ],
      out_specs=pl.BlockSpec((block, dout), lambda i: (i, 0)),
      out_shape=jax.ShapeDtypeStruct((n, dout), _f32),
  )(pa, pb, gwa, gwb, gb)


def _global_res_body(pa_ref, pb_ref, x_ref, gwa_ref, gwb_ref, gb_ref,
                     scw_ref, scb_ref, o_ref):
  a = pa_ref[0] + pa_ref[1]
  b = pb_ref[0] + pb_ref[1]
  h = a @ gwa_ref[...] + b @ gwb_ref[...] + gb_ref[...]
  h = jnp.maximum(h, 0.0)
  sc = x_ref[...] @ scw_ref[...] + scb_ref[...]
  o_ref[...] = jnp.maximum(h + sc, 0.0)


def _tc_global_res(pa, pb, x, gwa, gwb, gb, scw, scb, block=1000):
  _, n, da = pa.shape
  db = pb.shape[2]
  d = x.shape[1]
  dout = gwa.shape[1]
  grid = n // block
  full = lambda i: (0, 0)
  return pl.pallas_call(
      _global_res_body,
      grid=(grid,),
      in_specs=[
          pl.BlockSpec((NC, block, da), lambda i: (0, i, 0)),
          pl.BlockSpec((NC, block, db), lambda i: (0, i, 0)),
          pl.BlockSpec((block, d), lambda i: (i, 0)),
          pl.BlockSpec((da, dout), full),
          pl.BlockSpec((db, dout), full),
          pl.BlockSpec((1, dout), full),
          pl.BlockSpec((d, dout), full),
          pl.BlockSpec((1, dout), full),
      ],
      out_specs=pl.BlockSpec((block, dout), lambda i: (i, 0)),
      out_shape=jax.ShapeDtypeStruct((n, dout), _f32),
  )(pa, pb, x, gwa, gwb, gb, scw, scb)


def _prep_pp(ppW):
  wpi = jnp.zeros((16, ppW.shape[1]), _f32).at[:3].set(ppW[0:3] + ppW[6:9])
  wpj = jnp.zeros((16, ppW.shape[1]), _f32).at[:3].set(ppW[3:6] - ppW[6:9])
  w9 = ppW[9:10]
  return wpi, wpj, w9


def kernel(x, pos, edge_index, ppW1, ppb1, aW1, ab1, gW1, gb1,
           ppW2, ppb2, aW2, ab2, gW2, gb2, scW, scb):
  n, d = x.shape
  src = edge_index[0]
  dst = edge_index[1]
  pos16 = jnp.zeros((n, 16), _f32).at[:, :3].set(pos)

  pj16, pi16 = _sc_gather_pos(pos16, src, dst)
  xj = _sc_gather1(x, src)

  wpi1, wpj1, w91 = _prep_pp(ppW1)
  ma1, mb1 = _tc_edge(xj, pi16, pj16, wpi1, wpj1, w91, ppb1[None, :],
                      aW1[:d], aW1[d:], ab1[None, :])
  pa1, pb1 = _sc_scatter(ma1, mb1, dst, n)
  h1 = _tc_global(pa1, pb1, gW1[:d], gW1[d:], gb1[None, :])

  hj = _sc_gather1(h1, src)
  wpi2, wpj2, w92 = _prep_pp(ppW2)
  ma2, mb2 = _tc_edge(hj, pi16, pj16, wpi2, wpj2, w92, ppb2[None, :],
                      aW2[:d], aW2[d:], ab2[None, :])
  pa2, pb2 = _sc_scatter(ma2, mb2, dst, n)
  out = _tc_global_res(pa2, pb2, x, gW2[:d], gW2[d:], gb2[None, :],
                       scW, scb[None, :])
  return out
